# baseline (speedup 1.0000x reference)
import functools

import jax
import jax.numpy as jnp
from jax import lax
from jax.experimental import pallas as pl
from jax.experimental.pallas import tpu as pltpu
from jax.experimental.pallas import tpu_sc as plsc

_NC = 2
_NS = 16
_NW = _NC * _NS
_L = 16
_CH = 4
_NSLOT = 2
_LOOK = max(1, _NSLOT - 2)


def kernel(inputs, pos_table):
    B, S, D = inputs.shape
    rows_per_w = S // _NW
    chunks = rows_per_w // _CH
    mesh = plsc.VectorSubcoreMesh(core_axis_name="c", subcore_axis_name="s")

    @functools.partial(
        pl.kernel,
        out_type=jax.ShapeDtypeStruct((B, S, D), jnp.float32),
        mesh=mesh,
        scratch_types=(
            [pltpu.VMEM((_CH, D), jnp.float32) for _ in range(3 * _NSLOT)]
            + [pltpu.SemaphoreType.DMA for _ in range(2 * _NSLOT)]
        ),
    )
    def k(in_hbm, pos_hbm, out_hbm, *scr):
        pv = scr[0:_NSLOT]
        x0 = scr[_NSLOT:2 * _NSLOT]
        x1 = scr[2 * _NSLOT:3 * _NSLOT]
        isem = scr[3 * _NSLOT:4 * _NSLOT]
        osem = scr[4 * _NSLOT:5 * _NSLOT]

        wid = lax.axis_index("s") * _NC + lax.axis_index("c")
        rbase = wid * rows_per_w

        def fire_in(i, sl):
            r0 = rbase + i * _CH
            pltpu.async_copy(pos_hbm.at[pl.ds(r0, _CH)], pv[sl], isem[sl])
            pltpu.async_copy(in_hbm.at[0, pl.ds(r0, _CH)], x0[sl], isem[sl])
            pltpu.async_copy(in_hbm.at[1, pl.ds(r0, _CH)], x1[sl], isem[sl])

        def drain_in(sl):
            pltpu.make_async_copy(pos_hbm.at[pl.ds(0, _CH)], pv[sl], isem[sl]).wait()
            pltpu.make_async_copy(pos_hbm.at[pl.ds(0, _CH)], x0[sl], isem[sl]).wait()
            pltpu.make_async_copy(pos_hbm.at[pl.ds(0, _CH)], x1[sl], isem[sl]).wait()

        def fire_out(i, sl):
            r0 = rbase + i * _CH
            pltpu.async_copy(x0[sl], out_hbm.at[0, pl.ds(r0, _CH)], osem[sl])
            pltpu.async_copy(x1[sl], out_hbm.at[1, pl.ds(r0, _CH)], osem[sl])

        def drain_out(sl):
            pltpu.make_async_copy(x0[sl], out_hbm.at[0, pl.ds(0, _CH)], osem[sl]).wait()
            pltpu.make_async_copy(x1[sl], out_hbm.at[1, pl.ds(0, _CH)], osem[sl]).wait()

        def compute(sl):
            xa, xb, pp = x0[sl], x1[sl], pv[sl]
            for r in range(_CH):
                @plsc.parallel_loop(0, D, _L, unroll=8)
                def _body(j, r=r):
                    pj = pp[r, pl.ds(j, _L)]
                    xa[r, pl.ds(j, _L)] = xa[r, pl.ds(j, _L)] + pj
                    xb[r, pl.ds(j, _L)] = xb[r, pl.ds(j, _L)] + pj

        for t in range(_LOOK):
            fire_in(t, t % _NSLOT)

        def step(g, carry):
            for sl in range(_NSLOT):
                i = g * _NSLOT + sl
                nsl = (sl + _LOOK) % _NSLOT

                @pl.when(i + _LOOK < chunks)
                def _fire():
                    @pl.when(i + _LOOK >= _NSLOT)
                    def _drain():
                        drain_out(nsl)
                    fire_in(i + _LOOK, nsl)

                drain_in(sl)
                compute(sl)
                fire_out(i, sl)
            return carry

        lax.fori_loop(0, chunks // _NSLOT, step, 0)
        for sl in range(_NSLOT):
            drain_out(sl)

    return k(inputs, pos_table)


# speedup vs baseline: 1.0275x; 1.0275x over previous
import functools

import jax
import jax.numpy as jnp
from jax import lax
from jax.experimental import pallas as pl
from jax.experimental.pallas import tpu as pltpu
from jax.experimental.pallas import tpu_sc as plsc

_NC = 2
_NS = 16
_NW = _NC * _NS
_L = 16
_CH = 1
_NSLOT = 8
_LOOK = max(1, _NSLOT - 2)


def kernel(inputs, pos_table):
    B, S, D = inputs.shape
    rows_per_w = S // _NW
    chunks = rows_per_w // _CH
    mesh = plsc.VectorSubcoreMesh(core_axis_name="c", subcore_axis_name="s")

    @functools.partial(
        pl.kernel,
        out_type=jax.ShapeDtypeStruct((B, S, D), jnp.float32),
        mesh=mesh,
        scratch_types=(
            [pltpu.VMEM((_CH, D), jnp.float32) for _ in range(3 * _NSLOT)]
            + [pltpu.SemaphoreType.DMA for _ in range(2 * _NSLOT)]
        ),
    )
    def k(in_hbm, pos_hbm, out_hbm, *scr):
        pv = scr[0:_NSLOT]
        x0 = scr[_NSLOT:2 * _NSLOT]
        x1 = scr[2 * _NSLOT:3 * _NSLOT]
        isem = scr[3 * _NSLOT:4 * _NSLOT]
        osem = scr[4 * _NSLOT:5 * _NSLOT]

        wid = lax.axis_index("s") * _NC + lax.axis_index("c")
        rbase = wid * rows_per_w

        def fire_in(i, sl):
            r0 = rbase + i * _CH
            pltpu.async_copy(pos_hbm.at[pl.ds(r0, _CH)], pv[sl], isem[sl])
            pltpu.async_copy(in_hbm.at[0, pl.ds(r0, _CH)], x0[sl], isem[sl])
            pltpu.async_copy(in_hbm.at[1, pl.ds(r0, _CH)], x1[sl], isem[sl])

        def drain_in(sl):
            pltpu.make_async_copy(pos_hbm.at[pl.ds(0, _CH)], pv[sl], isem[sl]).wait()
            pltpu.make_async_copy(pos_hbm.at[pl.ds(0, _CH)], x0[sl], isem[sl]).wait()
            pltpu.make_async_copy(pos_hbm.at[pl.ds(0, _CH)], x1[sl], isem[sl]).wait()

        def fire_out(i, sl):
            r0 = rbase + i * _CH
            pltpu.async_copy(x0[sl], out_hbm.at[0, pl.ds(r0, _CH)], osem[sl])
            pltpu.async_copy(x1[sl], out_hbm.at[1, pl.ds(r0, _CH)], osem[sl])

        def drain_out(sl):
            pltpu.make_async_copy(x0[sl], out_hbm.at[0, pl.ds(0, _CH)], osem[sl]).wait()
            pltpu.make_async_copy(x1[sl], out_hbm.at[1, pl.ds(0, _CH)], osem[sl]).wait()

        def compute(sl):
            xa, xb, pp = x0[sl], x1[sl], pv[sl]
            for r in range(_CH):
                @plsc.parallel_loop(0, D, _L, unroll=8)
                def _body(j, r=r):
                    pj = pp[r, pl.ds(j, _L)]
                    xa[r, pl.ds(j, _L)] = xa[r, pl.ds(j, _L)] + pj
                    xb[r, pl.ds(j, _L)] = xb[r, pl.ds(j, _L)] + pj

        for t in range(_LOOK):
            fire_in(t, t % _NSLOT)

        def step(g, carry):
            for sl in range(_NSLOT):
                i = g * _NSLOT + sl
                nsl = (sl + _LOOK) % _NSLOT

                @pl.when(i + _LOOK < chunks)
                def _fire():
                    @pl.when(i + _LOOK >= _NSLOT)
                    def _drain():
                        drain_out(nsl)
                    fire_in(i + _LOOK, nsl)

                drain_in(sl)
                compute(sl)
                fire_out(i, sl)
            return carry

        lax.fori_loop(0, chunks // _NSLOT, step, 0)
        for sl in range(_NSLOT):
            drain_out(sl)

    return k(inputs, pos_table)
